# baseline (speedup 1.0000x reference)
import jax
import jax.numpy as jnp
from jax import lax
from jax.experimental import pallas as pl
from jax.experimental.pallas import tpu as pltpu
from jax.experimental.pallas import tpu_sc as plsc

N = 10000
D = 128
E = 160000
NC = 2
NS = 16
CHUNK = 80
EPT = E // NS
EPC = EPT // CHUNK
G16 = CHUNK // 16

BROW = 1000
BL = N // BROW


def _sc_mesh():
    return plsc.VectorSubcoreMesh(
        core_axis_name="c", subcore_axis_name="s", num_cores=NC, num_subcores=NS
    )


def _stage_chunk(src_1d, start, dst_ref):
    for k in range(G16):
        dst_ref[pl.ds(k * 16, 16)] = src_1d[pl.ds(start + k * 16, 16)]


def _deg_body(src_hbm, dst_hbm, zeros_hbm, out_hbm,
              src_v, dst_v, idx_i, ones_v, acc, ssem):
    c = lax.axis_index("c")
    s = lax.axis_index("s")

    def fill(i, carry):
        ones_v[i // 8, pl.ds((i % 8) * 16, 16)] = jnp.ones((16,), jnp.float32)
        return carry

    lax.fori_loop(0, CHUNK * 8, fill, 0)

    base = c * E + s * EPT
    pltpu.sync_copy(src_hbm.at[pl.ds(base, EPT)], src_v)
    pltpu.sync_copy(dst_hbm.at[pl.ds(base, EPT)], dst_v)

    def drain_one():
        pltpu.make_async_copy(zeros_hbm.at[pl.ds(0, CHUNK)], ones_v, ssem).wait()

    def histogram(idx_v, out_base):
        @pl.when(s < 10)
        def _():
            pltpu.sync_copy(zeros_hbm, acc.at[pl.ds(s * 1000, 1000)])

        plsc.subcore_barrier()

        def body(j, carry):
            p = lax.rem(j, 4)
            for k in range(G16):
                idx_i[p, pl.ds(k * 16, 16)] = idx_v[pl.ds(j * CHUNK + k * 16, 16)]
            pltpu.async_copy(ones_v, acc.at[idx_i.at[p]], ssem, add=True)

            @pl.when(j >= 3)
            def _():
                drain_one()

            return carry

        lax.fori_loop(0, EPC, body, 0)
        drain_one()
        drain_one()
        drain_one()
        plsc.subcore_barrier()

        @pl.when(s < 10)
        def _():
            pltpu.sync_copy(acc.at[pl.ds(s * 1000, 1000)],
                            out_hbm.at[pl.ds(out_base + s * 1000, 1000)])

        plsc.subcore_barrier()

    histogram(src_v, c * 2 * N)
    histogram(dst_v, c * 2 * N + N)


_deg_call = pl.kernel(
    _deg_body,
    out_type=jax.ShapeDtypeStruct((4 * N, D), jnp.float32),
    mesh=_sc_mesh(),
    scratch_types=[
        pltpu.VMEM((EPT,), jnp.int32),
        pltpu.VMEM((EPT,), jnp.int32),
        pltpu.VMEM((4, CHUNK), jnp.int32),
        pltpu.VMEM((CHUNK, D), jnp.float32),
        pltpu.VMEM_SHARED((N, D), jnp.float32),
        pltpu.SemaphoreType.DMA,
    ],
)


NBUF = 3
HALF0 = 63
HALF1 = EPC - HALF0


def _agg_body(m_hbm, src_hbm, dst_hbm, zeros_hbm, out_hbm,
              src_v, dst_v, dst_i, rows, acc, gsem):
    c = lax.axis_index("c")
    s = lax.axis_index("s")

    @pl.when(s < 10)
    def _():
        pltpu.sync_copy(zeros_hbm, acc.at[pl.ds(s * 1000, 1000)])

    base = c * E + s * EPT
    pltpu.sync_copy(dst_hbm.at[pl.ds(base, EPT)], dst_v)
    plsc.subcore_barrier()

    def gather(j, p):
        pltpu.async_copy(m_hbm.at[src_v.at[pl.ds(j * CHUNK, CHUNK)]],
                         rows.at[p], gsem)

    def wait_gather(p):
        pltpu.make_async_copy(m_hbm.at[pl.ds(0, CHUNK)], rows.at[p], gsem).wait()

    def run_half(off, nchunks):
        gather(0, 0)
        gather(1, 1)

        def body(j, carry):
            p = lax.rem(j, NBUF)
            wait_gather(p)

            @pl.when(j + 2 < nchunks)
            def _():
                gather(j + 2, lax.rem(j + 2, NBUF))

            for k in range(G16):
                dst_i[p, pl.ds(k * 16, 16)] = \
                    dst_v[pl.ds((off + j) * CHUNK + k * 16, 16)]
            pltpu.sync_copy(rows.at[p], acc.at[dst_i.at[p]], add=True)
            return carry

        lax.fori_loop(0, nchunks, body, 0)

    pltpu.sync_copy(src_hbm.at[pl.ds(base, HALF0 * CHUNK)],
                    src_v.at[pl.ds(0, HALF0 * CHUNK)])
    run_half(0, HALF0)
    pltpu.sync_copy(src_hbm.at[pl.ds(base + HALF0 * CHUNK, HALF1 * CHUNK)],
                    src_v.at[pl.ds(0, HALF1 * CHUNK)])
    run_half(HALF0, HALF1)
    plsc.subcore_barrier()

    @pl.when(s < 10)
    def _():
        pltpu.sync_copy(acc.at[pl.ds(s * 1000, 1000)],
                        out_hbm.at[pl.ds(c * N + s * 1000, 1000)])


_agg_call = pl.kernel(
    _agg_body,
    out_type=jax.ShapeDtypeStruct((2 * N, D), jnp.float32),
    mesh=_sc_mesh(),
    scratch_types=[
        pltpu.VMEM((HALF0 * CHUNK,), jnp.int32),
        pltpu.VMEM((EPT,), jnp.int32),
        pltpu.VMEM((NBUF, CHUNK), jnp.int32),
        pltpu.VMEM((NBUF, CHUNK, D), jnp.float32),
        pltpu.VMEM_SHARED((N, D), jnp.float32),
        pltpu.SemaphoreType.DMA,
    ],
)


def _norm_from_deg(d):
    return jnp.where(d > 0.0, lax.rsqrt(jnp.maximum(d, 1.0)), 0.0)


def _l2n(h):
    nrm = jnp.sqrt(jnp.sum(h * h, axis=-1, keepdims=True))
    return h / jnp.maximum(nrm, 1e-12)


def _pre_body(x_ref, ds_ref, w_ref, m_ref):
    xn = x_ref[...] * _norm_from_deg(ds_ref[...])
    m_ref[...] = jnp.dot(xn, w_ref[0], preferred_element_type=jnp.float32,
                         precision=lax.Precision.HIGHEST)


_pre_call = pl.pallas_call(
    _pre_body,
    grid=(2, BL),
    in_specs=[
        pl.BlockSpec((BROW, D), lambda r, i: (i, 0)),
        pl.BlockSpec((BROW, 1), lambda r, i: (r * BL + i, 0)),
        pl.BlockSpec((1, D, D), lambda r, i: (r, 0, 0)),
    ],
    out_specs=pl.BlockSpec((BROW, D), lambda r, i: (r * BL + i, 0)),
    out_shape=jax.ShapeDtypeStruct((2 * N, D), jnp.float32),
)


def _layer_h(a0, a1, dd0, dd1, b):
    h = (a0 * _norm_from_deg(dd0) + a1 * _norm_from_deg(dd1) + b[0:1, :])
    h = jnp.maximum(h, 0.0)
    return _l2n(h)


def _mid_body(a0_ref, a1_ref, dd0_ref, dd1_ref, b_ref, ds_ref, w_ref, m_ref):
    h = _layer_h(a0_ref[...], a1_ref[...], dd0_ref[...], dd1_ref[...], b_ref[...])
    m_ref[...] = jnp.dot(h * _norm_from_deg(ds_ref[...]), w_ref[0],
                         preferred_element_type=jnp.float32,
                         precision=lax.Precision.HIGHEST)


_mid_call = pl.pallas_call(
    _mid_body,
    grid=(2, BL),
    in_specs=[
        pl.BlockSpec((BROW, D), lambda r, i: (i, 0)),
        pl.BlockSpec((BROW, D), lambda r, i: (BL + i, 0)),
        pl.BlockSpec((BROW, 1), lambda r, i: (i, 0)),
        pl.BlockSpec((BROW, 1), lambda r, i: (BL + i, 0)),
        pl.BlockSpec((8, D), lambda r, i: (0, 0)),
        pl.BlockSpec((BROW, 1), lambda r, i: (r * BL + i, 0)),
        pl.BlockSpec((1, D, D), lambda r, i: (r, 0, 0)),
    ],
    out_specs=pl.BlockSpec((BROW, D), lambda r, i: (r * BL + i, 0)),
    out_shape=jax.ShapeDtypeStruct((2 * N, D), jnp.float32),
)


def _post_body(a0_ref, a1_ref, dd0_ref, dd1_ref, b_ref, o_ref):
    h = _layer_h(a0_ref[...], a1_ref[...], dd0_ref[...], dd1_ref[...], b_ref[...])
    o_ref[...] = _l2n(h)


_post_call = pl.pallas_call(
    _post_body,
    grid=(BL,),
    in_specs=[
        pl.BlockSpec((BROW, D), lambda i: (i, 0)),
        pl.BlockSpec((BROW, D), lambda i: (BL + i, 0)),
        pl.BlockSpec((BROW, 1), lambda i: (i, 0)),
        pl.BlockSpec((BROW, 1), lambda i: (BL + i, 0)),
        pl.BlockSpec((8, D), lambda i: (0, 0)),
    ],
    out_specs=pl.BlockSpec((BROW, D), lambda i: (i, 0)),
    out_shape=jax.ShapeDtypeStruct((N, D), jnp.float32),
)


def kernel(x, edge_index_rel0, edge_index_rel1,
           W0_rel0, W0_rel1, b0, W1_rel0, W1_rel1, b1):
    s0, d0 = edge_index_rel0[0], edge_index_rel0[1]
    s1, d1 = edge_index_rel1[0], edge_index_rel1[1]
    deg_src = jnp.concatenate([s0, s1])
    deg_dst = jnp.concatenate([d0, d1])
    agg_src = jnp.concatenate([s0, s1 + N])
    agg_dst = deg_dst

    zerosD = jnp.zeros((1000, D), jnp.float32)

    degout = _deg_call(deg_src, deg_dst, zerosD)
    ds_cat = jnp.concatenate([degout[0:N, :1], degout[2 * N:3 * N, :1]], axis=0)
    dd_cat = jnp.concatenate([degout[N:2 * N, :1], degout[3 * N:4 * N, :1]], axis=0)

    W0s = jnp.stack([W0_rel0, W0_rel1])
    W1s = jnp.stack([W1_rel0, W1_rel1])
    b0_8 = jnp.broadcast_to(b0[None, :], (8, D))
    b1_8 = jnp.broadcast_to(b1[None, :], (8, D))

    m1 = _pre_call(x, ds_cat, W0s)
    agg1 = _agg_call(m1, agg_src, agg_dst, zerosD)
    m2 = _mid_call(agg1, agg1, dd_cat, dd_cat, b0_8, ds_cat, W1s)
    agg2 = _agg_call(m2, agg_src, agg_dst, zerosD)
    return _post_call(agg2, agg2, dd_cat, dd_cat, b1_8)


# speedup vs baseline: 1.0597x; 1.0597x over previous
import jax
import jax.numpy as jnp
from jax import lax
from jax.experimental import pallas as pl
from jax.experimental.pallas import tpu as pltpu
from jax.experimental.pallas import tpu_sc as plsc

N = 10000
D = 128
E = 160000
NC = 2
NS = 16
CHUNK = 80
EPT = E // NS
EPC = EPT // CHUNK
G16 = CHUNK // 16

BROW = 2000
BL = N // BROW


def _sc_mesh():
    return plsc.VectorSubcoreMesh(
        core_axis_name="c", subcore_axis_name="s", num_cores=NC, num_subcores=NS
    )


def _stage_chunk(src_1d, start, dst_ref):
    for k in range(G16):
        dst_ref[pl.ds(k * 16, 16)] = src_1d[pl.ds(start + k * 16, 16)]


def _deg_body(src_hbm, dst_hbm, zeros_hbm, out_hbm,
              src_v, dst_v, idx_i, ones_v, acc, ssem):
    c = lax.axis_index("c")
    s = lax.axis_index("s")

    def fill(i, carry):
        ones_v[i // 8, pl.ds((i % 8) * 16, 16)] = jnp.ones((16,), jnp.float32)
        return carry

    lax.fori_loop(0, CHUNK * 8, fill, 0)

    base = c * E + s * EPT
    pltpu.sync_copy(src_hbm.at[pl.ds(base, EPT)], src_v)
    pltpu.sync_copy(dst_hbm.at[pl.ds(base, EPT)], dst_v)

    def drain_one():
        pltpu.make_async_copy(zeros_hbm.at[pl.ds(0, CHUNK)], ones_v, ssem).wait()

    def histogram(idx_v, out_base):
        @pl.when(s < 10)
        def _():
            pltpu.sync_copy(zeros_hbm, acc.at[pl.ds(s * 1000, 1000)])

        plsc.subcore_barrier()

        def body(j, carry):
            p = lax.rem(j, 4)
            for k in range(G16):
                idx_i[p, pl.ds(k * 16, 16)] = idx_v[pl.ds(j * CHUNK + k * 16, 16)]
            pltpu.async_copy(ones_v, acc.at[idx_i.at[p]], ssem, add=True)

            @pl.when(j >= 3)
            def _():
                drain_one()

            return carry

        lax.fori_loop(0, EPC, body, 0)
        drain_one()
        drain_one()
        drain_one()
        plsc.subcore_barrier()

        @pl.when(s < 10)
        def _():
            pltpu.sync_copy(acc.at[pl.ds(s * 1000, 1000)],
                            out_hbm.at[pl.ds(out_base + s * 1000, 1000)])

        plsc.subcore_barrier()

    histogram(src_v, c * 2 * N)
    histogram(dst_v, c * 2 * N + N)


_deg_call = pl.kernel(
    _deg_body,
    out_type=jax.ShapeDtypeStruct((4 * N, D), jnp.float32),
    mesh=_sc_mesh(),
    scratch_types=[
        pltpu.VMEM((EPT,), jnp.int32),
        pltpu.VMEM((EPT,), jnp.int32),
        pltpu.VMEM((4, CHUNK), jnp.int32),
        pltpu.VMEM((CHUNK, D), jnp.float32),
        pltpu.VMEM_SHARED((N, D), jnp.float32),
        pltpu.SemaphoreType.DMA,
    ],
)


NBUF = 3
HALF0 = 63
HALF1 = EPC - HALF0


def _agg_body(m_hbm, src_hbm, dst_hbm, zeros_hbm, out_hbm,
              src_v, dst_v, dst_i, rows, acc, gsem):
    c = lax.axis_index("c")
    s = lax.axis_index("s")

    @pl.when(s < 10)
    def _():
        pltpu.sync_copy(zeros_hbm, acc.at[pl.ds(s * 1000, 1000)])

    base = c * E + s * EPT
    pltpu.sync_copy(dst_hbm.at[pl.ds(base, EPT)], dst_v)
    plsc.subcore_barrier()

    def gather(j, p):
        pltpu.async_copy(m_hbm.at[src_v.at[pl.ds(j * CHUNK, CHUNK)]],
                         rows.at[p], gsem)

    def wait_gather(p):
        pltpu.make_async_copy(m_hbm.at[pl.ds(0, CHUNK)], rows.at[p], gsem).wait()

    def run_half(off, nchunks):
        gather(0, 0)
        gather(1, 1)

        def body(j, carry):
            p = lax.rem(j, NBUF)
            wait_gather(p)

            @pl.when(j + 2 < nchunks)
            def _():
                gather(j + 2, lax.rem(j + 2, NBUF))

            for k in range(G16):
                dst_i[p, pl.ds(k * 16, 16)] = \
                    dst_v[pl.ds((off + j) * CHUNK + k * 16, 16)]
            pltpu.sync_copy(rows.at[p], acc.at[dst_i.at[p]], add=True)
            return carry

        lax.fori_loop(0, nchunks, body, 0)

    pltpu.sync_copy(src_hbm.at[pl.ds(base, HALF0 * CHUNK)],
                    src_v.at[pl.ds(0, HALF0 * CHUNK)])
    run_half(0, HALF0)
    pltpu.sync_copy(src_hbm.at[pl.ds(base + HALF0 * CHUNK, HALF1 * CHUNK)],
                    src_v.at[pl.ds(0, HALF1 * CHUNK)])
    run_half(HALF0, HALF1)
    plsc.subcore_barrier()

    @pl.when(s < 10)
    def _():
        pltpu.sync_copy(acc.at[pl.ds(s * 1000, 1000)],
                        out_hbm.at[pl.ds(c * N + s * 1000, 1000)])


_agg_call = pl.kernel(
    _agg_body,
    out_type=jax.ShapeDtypeStruct((2 * N, D), jnp.float32),
    mesh=_sc_mesh(),
    scratch_types=[
        pltpu.VMEM((HALF0 * CHUNK,), jnp.int32),
        pltpu.VMEM((EPT,), jnp.int32),
        pltpu.VMEM((NBUF, CHUNK), jnp.int32),
        pltpu.VMEM((NBUF, CHUNK, D), jnp.float32),
        pltpu.VMEM_SHARED((N, D), jnp.float32),
        pltpu.SemaphoreType.DMA,
    ],
)


def _norm_from_deg(d):
    return jnp.where(d > 0.0, lax.rsqrt(jnp.maximum(d, 1.0)), 0.0)


def _l2n(h):
    nrm = jnp.sqrt(jnp.sum(h * h, axis=-1, keepdims=True))
    return h / jnp.maximum(nrm, 1e-12)


def _pre_body(x_ref, ds_ref, w_ref, m_ref):
    xn = x_ref[...] * _norm_from_deg(ds_ref[...])
    m_ref[...] = jnp.dot(xn, w_ref[0], preferred_element_type=jnp.float32,
                         precision=lax.Precision.HIGHEST)


_pre_call = pl.pallas_call(
    _pre_body,
    grid=(2, BL),
    in_specs=[
        pl.BlockSpec((BROW, D), lambda r, i: (i, 0)),
        pl.BlockSpec((BROW, 1), lambda r, i: (r * BL + i, 0)),
        pl.BlockSpec((1, D, D), lambda r, i: (r, 0, 0)),
    ],
    out_specs=pl.BlockSpec((BROW, D), lambda r, i: (r * BL + i, 0)),
    out_shape=jax.ShapeDtypeStruct((2 * N, D), jnp.float32),
)


def _layer_h(a0, a1, dd0, dd1, b):
    h = (a0 * _norm_from_deg(dd0) + a1 * _norm_from_deg(dd1) + b[0:1, :])
    h = jnp.maximum(h, 0.0)
    return _l2n(h)


def _mid_body(a0_ref, a1_ref, dd0_ref, dd1_ref, b_ref, ds_ref, w_ref, m_ref):
    h = _layer_h(a0_ref[...], a1_ref[...], dd0_ref[...], dd1_ref[...], b_ref[...])
    m_ref[...] = jnp.dot(h * _norm_from_deg(ds_ref[...]), w_ref[0],
                         preferred_element_type=jnp.float32,
                         precision=lax.Precision.HIGHEST)


_mid_call = pl.pallas_call(
    _mid_body,
    grid=(2, BL),
    in_specs=[
        pl.BlockSpec((BROW, D), lambda r, i: (i, 0)),
        pl.BlockSpec((BROW, D), lambda r, i: (BL + i, 0)),
        pl.BlockSpec((BROW, 1), lambda r, i: (i, 0)),
        pl.BlockSpec((BROW, 1), lambda r, i: (BL + i, 0)),
        pl.BlockSpec((8, D), lambda r, i: (0, 0)),
        pl.BlockSpec((BROW, 1), lambda r, i: (r * BL + i, 0)),
        pl.BlockSpec((1, D, D), lambda r, i: (r, 0, 0)),
    ],
    out_specs=pl.BlockSpec((BROW, D), lambda r, i: (r * BL + i, 0)),
    out_shape=jax.ShapeDtypeStruct((2 * N, D), jnp.float32),
)


def _post_body(a0_ref, a1_ref, dd0_ref, dd1_ref, b_ref, o_ref):
    h = _layer_h(a0_ref[...], a1_ref[...], dd0_ref[...], dd1_ref[...], b_ref[...])
    o_ref[...] = _l2n(h)


_post_call = pl.pallas_call(
    _post_body,
    grid=(BL,),
    in_specs=[
        pl.BlockSpec((BROW, D), lambda i: (i, 0)),
        pl.BlockSpec((BROW, D), lambda i: (BL + i, 0)),
        pl.BlockSpec((BROW, 1), lambda i: (i, 0)),
        pl.BlockSpec((BROW, 1), lambda i: (BL + i, 0)),
        pl.BlockSpec((8, D), lambda i: (0, 0)),
    ],
    out_specs=pl.BlockSpec((BROW, D), lambda i: (i, 0)),
    out_shape=jax.ShapeDtypeStruct((N, D), jnp.float32),
)


def kernel(x, edge_index_rel0, edge_index_rel1,
           W0_rel0, W0_rel1, b0, W1_rel0, W1_rel1, b1):
    s0, d0 = edge_index_rel0[0], edge_index_rel0[1]
    s1, d1 = edge_index_rel1[0], edge_index_rel1[1]
    deg_src = jnp.concatenate([s0, s1])
    deg_dst = jnp.concatenate([d0, d1])
    agg_src = jnp.concatenate([s0, s1 + N])
    agg_dst = deg_dst

    zerosD = jnp.zeros((1000, D), jnp.float32)

    degout = _deg_call(deg_src, deg_dst, zerosD)
    ds_cat = jnp.concatenate([degout[0:N, :1], degout[2 * N:3 * N, :1]], axis=0)
    dd_cat = jnp.concatenate([degout[N:2 * N, :1], degout[3 * N:4 * N, :1]], axis=0)

    W0s = jnp.stack([W0_rel0, W0_rel1])
    W1s = jnp.stack([W1_rel0, W1_rel1])
    b0_8 = jnp.broadcast_to(b0[None, :], (8, D))
    b1_8 = jnp.broadcast_to(b1[None, :], (8, D))

    m1 = _pre_call(x, ds_cat, W0s)
    agg1 = _agg_call(m1, agg_src, agg_dst, zerosD)
    m2 = _mid_call(agg1, agg1, dd_cat, dd_cat, b0_8, ds_cat, W1s)
    agg2 = _agg_call(m2, agg_src, agg_dst, zerosD)
    return _post_call(agg2, agg2, dd_cat, dd_cat, b1_8)


# speedup vs baseline: 1.0729x; 1.0124x over previous
import jax
import jax.numpy as jnp
from jax import lax
from jax.experimental import pallas as pl
from jax.experimental.pallas import tpu as pltpu
from jax.experimental.pallas import tpu_sc as plsc

N = 10000
D = 128
E = 160000
NC = 2
NS = 16
CHUNK = 80
EPT = E // NS
EPC = EPT // CHUNK
G16 = CHUNK // 16

BROW = 2000
BL = N // BROW


def _sc_mesh():
    return plsc.VectorSubcoreMesh(
        core_axis_name="c", subcore_axis_name="s", num_cores=NC, num_subcores=NS
    )


def _stage_chunk(src_1d, start, dst_ref):
    for k in range(G16):
        dst_ref[pl.ds(k * 16, 16)] = src_1d[pl.ds(start + k * 16, 16)]


def _deg_body(src_hbm, dst_hbm, zeros_hbm, out_hbm,
              src_v, dst_v, idx_i, ones_v, acc, ssem):
    c = lax.axis_index("c")
    s = lax.axis_index("s")

    def fill(i, carry):
        ones_v[i // 8, pl.ds((i % 8) * 16, 16)] = jnp.ones((16,), jnp.float32)
        return carry

    lax.fori_loop(0, CHUNK * 8, fill, 0)

    base = c * E + s * EPT
    pltpu.sync_copy(src_hbm.at[pl.ds(base, EPT)], src_v)
    pltpu.sync_copy(dst_hbm.at[pl.ds(base, EPT)], dst_v)

    def drain_one():
        pltpu.make_async_copy(zeros_hbm.at[pl.ds(0, CHUNK)], ones_v, ssem).wait()

    def histogram(idx_v, out_base):
        @pl.when(s < 10)
        def _():
            pltpu.sync_copy(zeros_hbm, acc.at[pl.ds(s * 1000, 1000)])

        plsc.subcore_barrier()

        def body(j, carry):
            p = lax.rem(j, 4)
            for k in range(G16):
                idx_i[p, pl.ds(k * 16, 16)] = idx_v[pl.ds(j * CHUNK + k * 16, 16)]
            pltpu.async_copy(ones_v, acc.at[idx_i.at[p]], ssem, add=True)

            @pl.when(j >= 3)
            def _():
                drain_one()

            return carry

        lax.fori_loop(0, EPC, body, 0)
        drain_one()
        drain_one()
        drain_one()
        plsc.subcore_barrier()

        @pl.when(s < 10)
        def _():
            pltpu.sync_copy(acc.at[pl.ds(s * 1000, 1000)],
                            out_hbm.at[pl.ds(out_base + s * 1000, 1000)])

        plsc.subcore_barrier()

    histogram(src_v, c * 2 * N)
    histogram(dst_v, c * 2 * N + N)


_deg_call = pl.kernel(
    _deg_body,
    out_type=jax.ShapeDtypeStruct((4 * N, D), jnp.float32),
    mesh=_sc_mesh(),
    scratch_types=[
        pltpu.VMEM((EPT,), jnp.int32),
        pltpu.VMEM((EPT,), jnp.int32),
        pltpu.VMEM((4, CHUNK), jnp.int32),
        pltpu.VMEM((CHUNK, D), jnp.float32),
        pltpu.VMEM_SHARED((N, D), jnp.float32),
        pltpu.SemaphoreType.DMA,
    ],
)


NBUF = 4
PARTS = (42, 42, 41)


def _agg_body(m_hbm, src_hbm, dst_hbm, zeros_hbm, out_hbm,
              src_v, dst_v, dst_i, rows, acc, gsem):
    c = lax.axis_index("c")
    s = lax.axis_index("s")

    @pl.when(s < 10)
    def _():
        pltpu.sync_copy(zeros_hbm, acc.at[pl.ds(s * 1000, 1000)])

    base = c * E + s * EPT
    plsc.subcore_barrier()

    def gather(j, p):
        pltpu.async_copy(m_hbm.at[src_v.at[pl.ds(j * CHUNK, CHUNK)]],
                         rows.at[p], gsem)

    def wait_gather(p):
        pltpu.make_async_copy(m_hbm.at[pl.ds(0, CHUNK)], rows.at[p], gsem).wait()

    def run_part(nchunks):
        gather(0, 0)
        gather(1, 1)
        gather(2, 2)

        def body(j, carry):
            p = lax.rem(j, NBUF)
            wait_gather(p)

            @pl.when(j + 3 < nchunks)
            def _():
                gather(j + 3, lax.rem(j + 3, NBUF))

            for k in range(G16):
                dst_i[p, pl.ds(k * 16, 16)] = \
                    dst_v[pl.ds(j * CHUNK + k * 16, 16)]
            pltpu.sync_copy(rows.at[p], acc.at[dst_i.at[p]], add=True)
            return carry

        lax.fori_loop(0, nchunks, body, 0)

    off = 0
    for nchunks in PARTS:
        pltpu.sync_copy(src_hbm.at[pl.ds(base + off, nchunks * CHUNK)],
                        src_v.at[pl.ds(0, nchunks * CHUNK)])
        pltpu.sync_copy(dst_hbm.at[pl.ds(base + off, nchunks * CHUNK)],
                        dst_v.at[pl.ds(0, nchunks * CHUNK)])
        run_part(nchunks)
        off += nchunks * CHUNK
    plsc.subcore_barrier()

    @pl.when(s < 10)
    def _():
        pltpu.sync_copy(acc.at[pl.ds(s * 1000, 1000)],
                        out_hbm.at[pl.ds(c * N + s * 1000, 1000)])


_agg_call = pl.kernel(
    _agg_body,
    out_type=jax.ShapeDtypeStruct((2 * N, D), jnp.float32),
    mesh=_sc_mesh(),
    scratch_types=[
        pltpu.VMEM((PARTS[0] * CHUNK,), jnp.int32),
        pltpu.VMEM((PARTS[0] * CHUNK,), jnp.int32),
        pltpu.VMEM((NBUF, CHUNK), jnp.int32),
        pltpu.VMEM((NBUF, CHUNK, D), jnp.float32),
        pltpu.VMEM_SHARED((N, D), jnp.float32),
        pltpu.SemaphoreType.DMA,
    ],
)


def _norm_from_deg(d):
    return jnp.where(d > 0.0, lax.rsqrt(jnp.maximum(d, 1.0)), 0.0)


def _l2n(h):
    nrm = jnp.sqrt(jnp.sum(h * h, axis=-1, keepdims=True))
    return h / jnp.maximum(nrm, 1e-12)


def _pre_body(x_ref, ds_ref, w_ref, m_ref):
    xn = x_ref[...] * _norm_from_deg(ds_ref[...])
    m_ref[...] = jnp.dot(xn, w_ref[0], preferred_element_type=jnp.float32,
                         precision=lax.Precision.HIGHEST)


_pre_call = pl.pallas_call(
    _pre_body,
    grid=(2, BL),
    in_specs=[
        pl.BlockSpec((BROW, D), lambda r, i: (i, 0)),
        pl.BlockSpec((BROW, 1), lambda r, i: (r * BL + i, 0)),
        pl.BlockSpec((1, D, D), lambda r, i: (r, 0, 0)),
    ],
    out_specs=pl.BlockSpec((BROW, D), lambda r, i: (r * BL + i, 0)),
    out_shape=jax.ShapeDtypeStruct((2 * N, D), jnp.float32),
)


def _layer_h(a0, a1, dd0, dd1, b):
    h = (a0 * _norm_from_deg(dd0) + a1 * _norm_from_deg(dd1) + b[0:1, :])
    h = jnp.maximum(h, 0.0)
    return _l2n(h)


def _mid_body(a0_ref, a1_ref, dd0_ref, dd1_ref, b_ref, ds_ref, w_ref, m_ref):
    h = _layer_h(a0_ref[...], a1_ref[...], dd0_ref[...], dd1_ref[...], b_ref[...])
    m_ref[...] = jnp.dot(h * _norm_from_deg(ds_ref[...]), w_ref[0],
                         preferred_element_type=jnp.float32,
                         precision=lax.Precision.HIGHEST)


_mid_call = pl.pallas_call(
    _mid_body,
    grid=(2, BL),
    in_specs=[
        pl.BlockSpec((BROW, D), lambda r, i: (i, 0)),
        pl.BlockSpec((BROW, D), lambda r, i: (BL + i, 0)),
        pl.BlockSpec((BROW, 1), lambda r, i: (i, 0)),
        pl.BlockSpec((BROW, 1), lambda r, i: (BL + i, 0)),
        pl.BlockSpec((8, D), lambda r, i: (0, 0)),
        pl.BlockSpec((BROW, 1), lambda r, i: (r * BL + i, 0)),
        pl.BlockSpec((1, D, D), lambda r, i: (r, 0, 0)),
    ],
    out_specs=pl.BlockSpec((BROW, D), lambda r, i: (r * BL + i, 0)),
    out_shape=jax.ShapeDtypeStruct((2 * N, D), jnp.float32),
)


def _post_body(a0_ref, a1_ref, dd0_ref, dd1_ref, b_ref, o_ref):
    h = _layer_h(a0_ref[...], a1_ref[...], dd0_ref[...], dd1_ref[...], b_ref[...])
    o_ref[...] = _l2n(h)


_post_call = pl.pallas_call(
    _post_body,
    grid=(BL,),
    in_specs=[
        pl.BlockSpec((BROW, D), lambda i: (i, 0)),
        pl.BlockSpec((BROW, D), lambda i: (BL + i, 0)),
        pl.BlockSpec((BROW, 1), lambda i: (i, 0)),
        pl.BlockSpec((BROW, 1), lambda i: (BL + i, 0)),
        pl.BlockSpec((8, D), lambda i: (0, 0)),
    ],
    out_specs=pl.BlockSpec((BROW, D), lambda i: (i, 0)),
    out_shape=jax.ShapeDtypeStruct((N, D), jnp.float32),
)


def kernel(x, edge_index_rel0, edge_index_rel1,
           W0_rel0, W0_rel1, b0, W1_rel0, W1_rel1, b1):
    s0, d0 = edge_index_rel0[0], edge_index_rel0[1]
    s1, d1 = edge_index_rel1[0], edge_index_rel1[1]
    deg_src = jnp.concatenate([s0, s1])
    deg_dst = jnp.concatenate([d0, d1])
    agg_src = jnp.concatenate([s0, s1 + N])
    agg_dst = deg_dst

    zerosD = jnp.zeros((1000, D), jnp.float32)

    degout = _deg_call(deg_src, deg_dst, zerosD)
    ds_cat = jnp.concatenate([degout[0:N, :1], degout[2 * N:3 * N, :1]], axis=0)
    dd_cat = jnp.concatenate([degout[N:2 * N, :1], degout[3 * N:4 * N, :1]], axis=0)

    W0s = jnp.stack([W0_rel0, W0_rel1])
    W1s = jnp.stack([W1_rel0, W1_rel1])
    b0_8 = jnp.broadcast_to(b0[None, :], (8, D))
    b1_8 = jnp.broadcast_to(b1[None, :], (8, D))

    m1 = _pre_call(x, ds_cat, W0s)
    agg1 = _agg_call(m1, agg_src, agg_dst, zerosD)
    m2 = _mid_call(agg1, agg1, dd_cat, dd_cat, b0_8, ds_cat, W1s)
    agg2 = _agg_call(m2, agg_src, agg_dst, zerosD)
    return _post_call(agg2, agg2, dd_cat, dd_cat, b1_8)
